# baseline (speedup 1.0000x reference)
import functools

import jax
import jax.numpy as jnp
from jax import lax
from jax.experimental import pallas as pl
from jax.experimental.pallas import tpu as pltpu
from jax.experimental.pallas import tpu_sc as plsc

IN_C = 128
SPIRAL = 32
OUT_C = 128
N_PTS = 10000

G = 4
SG = SPIRAL // G

_info = plsc.get_sparse_core_info()
NC = _info.num_cores
NS = _info.num_subcores
L = _info.num_lanes
NW = NC * NS

P = 16
ROWS = P * SG
NCHUNK = N_PTS // P
NIT = (NCHUNK + NW - 1) // NW
VPR = OUT_C // L


def _mm_body(x_ref, w_ref, y_ref):
    y_ref[0] = jnp.dot(x_ref[...], w_ref[0],
                       preferred_element_type=jnp.float32)


def _matmul(x2d, wg):
    return pl.pallas_call(
        _mm_body,
        grid=(SG,),
        in_specs=[
            pl.BlockSpec((N_PTS, IN_C), lambda s: (0, 0)),
            pl.BlockSpec((1, IN_C, OUT_C), lambda s: (s, 0, 0)),
        ],
        out_specs=pl.BlockSpec((1, N_PTS, OUT_C), lambda s: (s, 0, 0)),
        out_shape=jax.ShapeDtypeStruct((SG, N_PTS, OUT_C), jnp.float32),
    )(x2d, wg)


_mesh = plsc.VectorSubcoreMesh(core_axis_name="c", subcore_axis_name="s")


@functools.partial(
    pl.kernel,
    out_type=jax.ShapeDtypeStruct((N_PTS, OUT_C), jnp.float32),
    mesh=_mesh,
    scratch_types=[
        pltpu.VMEM((ROWS,), jnp.int32),
        pltpu.VMEM((ROWS,), jnp.int32),
        pltpu.VMEM((1, ROWS), jnp.int32),
        pltpu.VMEM((1, ROWS), jnp.int32),
        pltpu.VMEM((ROWS, OUT_C), jnp.float32),
        pltpu.VMEM((ROWS, OUT_C), jnp.float32),
        pltpu.VMEM((P, OUT_C), jnp.float32),
        pltpu.SemaphoreType.DMA,
        pltpu.SemaphoreType.DMA,
        pltpu.SemaphoreType.DMA,
        pltpu.SemaphoreType.DMA,
    ],
)
def _sc_gather(y_hbm, adj_hbm, out_hbm,
               adj0, adj1, idx0, idx1, rows0, rows1, out_v,
               sema0, sema1, semr0, semr1):
    wid = lax.axis_index("s") * NC + lax.axis_index("c")
    svec = (lax.iota(jnp.int32, L) % SG) * N_PTS

    def chunk_of(i):
        return jnp.minimum(wid + i * NW, NCHUNK - 1)

    def adj_cp(i, adj_v, sema):
        c = chunk_of(i)
        return pltpu.make_async_copy(
            adj_hbm.at[pl.ds(c * ROWS, ROWS)], adj_v, sema)

    def gather_cp(idx_v, rows_v, semr):
        return pltpu.make_async_copy(y_hbm.at[idx_v.at[0]], rows_v, semr)

    def build_idx(adj_v, idx_v):
        for v in range(ROWS // L):
            idx_v[0, pl.ds(v * L, L)] = adj_v[pl.ds(v * L, L)] + svec

    def step(i, cur, nxt):
        (c_adj, c_idx, c_rows, c_sema, c_semr) = cur
        (n_adj, n_idx, n_rows, n_sema, n_semr) = nxt

        @pl.when(i + 1 < NIT)
        def _():
            adj_cp(i + 1, n_adj, n_sema).wait()
            build_idx(n_adj, n_idx)
            gather_cp(n_idx, n_rows, n_semr).start()

        @pl.when(i + 2 < NIT)
        def _():
            adj_cp(i + 2, c_adj, c_sema).start()

        gather_cp(c_idx, c_rows, c_semr).wait()

        c = chunk_of(i)
        for p in range(P):
            accs = tuple(jnp.zeros((L,), jnp.float32) for _ in range(VPR))

            def s_body(s, acc):
                r = p * SG + s
                return tuple(a + c_rows[r, pl.ds(v * L, L)]
                             for v, a in enumerate(acc))

            accs = lax.fori_loop(0, SG, s_body, accs)
            for v in range(VPR):
                out_v[p, pl.ds(v * L, L)] = accs[v]

        @pl.when(wid + i * NW < NCHUNK)
        def _():
            pltpu.sync_copy(out_v, out_hbm.at[pl.ds(c * P, P)])

    slot0 = (adj0, idx0, rows0, sema0, semr0)
    slot1 = (adj1, idx1, rows1, sema1, semr1)

    adj_cp(0, adj0, sema0).start()
    adj_cp(0, adj0, sema0).wait()
    build_idx(adj0, idx0)
    gather_cp(idx0, rows0, semr0).start()
    adj_cp(1, adj1, sema1).start()

    def pair_body(g, carry):
        step(2 * g, slot0, slot1)
        step(2 * g + 1, slot1, slot0)
        return carry

    lax.fori_loop(0, NIT // 2, pair_body, 0)


def _comb_body(p0_ref, p1_ref, p2_ref, p3_ref, b_ref, o_ref):
    i = pl.program_id(0)
    z = (p0_ref[...] + p1_ref[...] + p2_ref[...] + p3_ref[...]
         + b_ref[...])
    y = jnp.where(z > 0.0, z, jnp.exp(jnp.minimum(z, 0.0)) - 1.0)
    rows = jax.lax.broadcasted_iota(jnp.int32, y.shape, 0) + i * y.shape[0]
    o_ref[...] = jnp.where(rows == N_PTS - 1, 0.0, y)


def _combine(parts, b2d):
    BM = 1000
    bs = pl.BlockSpec((BM, OUT_C), lambda i: (i, 0))
    return pl.pallas_call(
        _comb_body,
        grid=(N_PTS // BM,),
        in_specs=[bs, bs, bs, bs,
                  pl.BlockSpec((1, OUT_C), lambda i: (0, 0))],
        out_specs=bs,
        out_shape=jax.ShapeDtypeStruct((N_PTS, OUT_C), jnp.float32),
    )(*parts, b2d)


def kernel(x, spiral_adj, W, b):
    x2d = x.reshape(N_PTS, IN_C).astype(jnp.bfloat16)
    adjg = (spiral_adj.reshape(N_PTS, G, SG).astype(jnp.int32)
            .transpose(1, 0, 2).reshape(G, N_PTS * SG))
    wmat4 = (W.reshape(OUT_C, G, SG, IN_C).transpose(1, 2, 3, 0)
             .astype(jnp.bfloat16))
    parts = []
    for g in range(G):
        yg = _matmul(x2d, wmat4[g])
        parts.append(_sc_gather(yg.reshape(SG * N_PTS, OUT_C), adjg[g]))
    out2d = _combine(parts, b.reshape(1, OUT_C))
    return out2d.reshape(1, N_PTS, OUT_C)


# speedup vs baseline: 2.3094x; 2.3094x over previous
import functools

import jax
import jax.numpy as jnp
from jax import lax
from jax.experimental import pallas as pl
from jax.experimental.pallas import tpu as pltpu
from jax.experimental.pallas import tpu_sc as plsc

IN_C = 128
SPIRAL = 32
OUT_C = 128
N_PTS = 10000

_info = plsc.get_sparse_core_info()
NC = _info.num_cores
NS = _info.num_subcores
L = _info.num_lanes
NW = NC * NS

PC = 16
RPC = PC * SPIRAL
NCHUNK = N_PTS // PC
NIT = (NCHUNK + NW - 1) // NW
VPR = OUT_C // L


def _mm_body(x_ref, w_ref, y_ref):
    y_ref[0] = jnp.dot(x_ref[...], w_ref[0],
                       preferred_element_type=jnp.float32)


def _matmul(x2d, wmat3):
    return pl.pallas_call(
        _mm_body,
        grid=(SPIRAL,),
        in_specs=[
            pl.BlockSpec((N_PTS, IN_C), lambda s: (0, 0)),
            pl.BlockSpec((1, IN_C, OUT_C), lambda s: (s, 0, 0)),
        ],
        out_specs=pl.BlockSpec((1, N_PTS, OUT_C), lambda s: (s, 0, 0)),
        out_shape=jax.ShapeDtypeStruct((SPIRAL, N_PTS, OUT_C), jnp.float32),
    )(x2d, wmat3)


_mesh = plsc.VectorSubcoreMesh(core_axis_name="c", subcore_axis_name="s")


@functools.partial(
    pl.kernel,
    out_type=jax.ShapeDtypeStruct((N_PTS, OUT_C), jnp.float32),
    mesh=_mesh,
    scratch_types=[
        pltpu.VMEM((RPC,), jnp.int32),
        pltpu.VMEM((RPC,), jnp.int32),
        pltpu.VMEM((SPIRAL, PC), jnp.int32),
        pltpu.VMEM((SPIRAL, PC), jnp.int32),
        pltpu.VMEM((PC, OUT_C), jnp.float32),
        pltpu.VMEM((PC, OUT_C), jnp.float32),
        pltpu.VMEM((OUT_C,), jnp.float32),
        pltpu.SemaphoreType.DMA,
        pltpu.SemaphoreType.DMA,
        pltpu.SemaphoreType.DMA,
        pltpu.SemaphoreType.DMA,
        pltpu.SemaphoreType.DMA,
        pltpu.SemaphoreType.DMA,
    ],
)
def _sc_gather(y_hbm, adj_hbm, b_hbm, out_hbm,
               adj0, adj1, idx0, idx1, acc0, acc1, bias_v,
               sema0, sema1, semr0, semr1, semo0, semo1):
    wid = lax.axis_index("s") * NC + lax.axis_index("c")
    pltpu.sync_copy(b_hbm, bias_v)

    def chunk_of(i):
        return jnp.minimum(wid + i * NW, NCHUNK - 1)

    def adj_cp(i, adj_v, sema):
        c = chunk_of(i)
        return pltpu.make_async_copy(
            adj_hbm.at[pl.ds(c * RPC, RPC)], adj_v, sema)

    def out_cp(i, acc_v, semo):
        c = chunk_of(i)
        return pltpu.make_async_copy(
            acc_v, out_hbm.at[pl.ds(c * PC, PC)], semo)

    def stage(adj_v, idx_v, acc_v, semr):
        def init_body(p, carry):
            for v in range(VPR):
                acc_v[p, pl.ds(v * L, L)] = bias_v[pl.ds(v * L, L)]
            return carry

        lax.fori_loop(0, PC, init_body, 0)

        for s in range(SPIRAL):
            av = adj_v[pl.ds(s * PC, PC)]
            idx_v[s, pl.ds(0, PC)] = av + s * N_PTS
            pltpu.async_copy(y_hbm.at[idx_v.at[s]], acc_v, semr, add=True)

    def drain(idx_v, acc_v, semr):
        def d_body(s, carry):
            pltpu.make_async_copy(y_hbm.at[idx_v.at[0]], acc_v, semr).wait()
            return carry

        lax.fori_loop(0, SPIRAL, d_body, 0)

    def elu_zero(i, acc_v):
        c = chunk_of(i)

        def e_body(p, carry):
            for v in range(VPR):
                z = acc_v[p, pl.ds(v * L, L)]
                acc_v[p, pl.ds(v * L, L)] = jnp.where(
                    z > 0.0, z, jnp.exp(jnp.minimum(z, 0.0)) - 1.0)
            return carry

        lax.fori_loop(0, PC, e_body, 0)

        @pl.when(c == NCHUNK - 1)
        def _():
            zero = jnp.zeros((L,), jnp.float32)
            for v in range(VPR):
                acc_v[PC - 1, pl.ds(v * L, L)] = zero

    def valid(i):
        return wid + i * NW < NCHUNK

    def step(i, cur, nxt):
        (c_adj, c_idx, c_acc, c_sema, c_semr, c_semo) = cur
        (n_adj, n_idx, n_acc, n_sema, n_semr, n_semo) = nxt

        @pl.when(i + 1 < NIT)
        def _():
            @pl.when((i >= 1) & valid(i - 1))
            def _():
                out_cp(i - 1, n_acc, n_semo).wait()

            adj_cp(i + 1, n_adj, n_sema).wait()
            stage(n_adj, n_idx, n_acc, n_semr)

        @pl.when(i + 2 < NIT)
        def _():
            adj_cp(i + 2, c_adj, c_sema).start()

        drain(c_idx, c_acc, c_semr)
        elu_zero(i, c_acc)

        @pl.when(valid(i))
        def _():
            out_cp(i, c_acc, c_semo).start()

    slot0 = (adj0, idx0, acc0, sema0, semr0, semo0)
    slot1 = (adj1, idx1, acc1, sema1, semr1, semo1)

    adj_cp(0, adj0, sema0).start()
    adj_cp(0, adj0, sema0).wait()
    stage(adj0, idx0, acc0, semr0)
    adj_cp(1, adj1, sema1).start()

    def pair_body(g, carry):
        step(2 * g, slot0, slot1)
        step(2 * g + 1, slot1, slot0)
        return carry

    lax.fori_loop(0, NIT // 2, pair_body, 0)

    @pl.when(valid(NIT - 2))
    def _():
        out_cp(NIT - 2, acc0 if (NIT - 2) % 2 == 0 else acc1,
               semo0 if (NIT - 2) % 2 == 0 else semo1).wait()

    @pl.when(valid(NIT - 1))
    def _():
        out_cp(NIT - 1, acc0 if (NIT - 1) % 2 == 0 else acc1,
               semo0 if (NIT - 1) % 2 == 0 else semo1).wait()


def kernel(x, spiral_adj, W, b):
    x2d = x.reshape(N_PTS, IN_C).astype(jnp.bfloat16)
    adj = (spiral_adj.reshape(NCHUNK, PC, SPIRAL).astype(jnp.int32)
           .transpose(0, 2, 1).reshape(N_PTS * SPIRAL))
    wmat3 = (W.reshape(OUT_C, SPIRAL, IN_C).transpose(1, 2, 0)
             .astype(jnp.bfloat16))
    y = _matmul(x2d, wmat3)
    yr = y.reshape(SPIRAL * N_PTS, OUT_C)
    out2d = _sc_gather(yr, adj, b)
    return out2d.reshape(1, N_PTS, OUT_C)
